# baseline (speedup 1.0000x reference)
import jax
import jax.numpy as jnp
from jax.experimental import pallas as pl
from jax.experimental.pallas import tpu as pltpu

N = 8192
D = 128
BM = 256
NB = N // BM


def _mega_kernel(a_ref, x_ref, w_ref, o_ref, a16_ref,
                 ab, y_scr, sem_w, sem_r):
    l = pl.program_id(0)
    i = pl.program_id(1)
    slot = jax.lax.rem(i, 2)
    nslot = jax.lax.rem(i + 1, 2)

    def wcopy(s, blk):
        return pltpu.make_async_copy(
            ab.at[s], a16_ref.at[pl.ds(blk * BM, BM)], sem_w.at[s])

    def rcopy(s, blk):
        return pltpu.make_async_copy(
            a16_ref.at[pl.ds(blk * BM, BM)], ab.at[s], sem_r.at[s])

    @pl.when((l == 0) & (i == 0))
    def _y0_prologue():
        y0 = jnp.dot(x_ref[...], w_ref[0],
                     preferred_element_type=jnp.float32)
        y_scr[0] = y0.astype(jnp.bfloat16)

    @pl.when(l == 0)
    def _phase0():
        @pl.when(i >= 2)
        def _():
            wcopy(slot, i - 2).wait()
        a16 = a_ref[...].astype(jnp.bfloat16)
        ab[slot] = a16
        h = jnp.maximum(
            jnp.dot(a16, y_scr[0], preferred_element_type=jnp.float32), 0.0)
        o_ref[...] = h
        y1 = jnp.dot(h, w_ref[1], preferred_element_type=jnp.float32)
        y_scr[1, pl.ds(i * BM, BM), :] = y1.astype(jnp.bfloat16)
        wcopy(slot, i).start()

    @pl.when(l >= 1)
    def _phase12():
        @pl.when((l == 1) & (i == 0))
        def _drain_and_first_read():
            wcopy(jax.lax.rem(NB - 2, 2), NB - 2).wait()
            wcopy(jax.lax.rem(NB - 1, 2), NB - 1).wait()
            rcopy(0, 0).start()

        @pl.when(i + 1 < NB)
        def _prefetch_next():
            rcopy(nslot, i + 1).start()

        @pl.when((i + 1 == NB) & (l == 1))
        def _prefetch_next_phase():
            rcopy(nslot, 0).start()

        rcopy(slot, i).wait()
        yv = y_scr[jnp.where(l == 1, 1, 0)]
        h = jnp.maximum(
            jnp.dot(ab[slot], yv, preferred_element_type=jnp.float32), 0.0)
        o_ref[...] = h

        @pl.when(l == 1)
        def _epilogue():
            y2 = jnp.dot(h, w_ref[2], preferred_element_type=jnp.float32)
            y_scr[0, pl.ds(i * BM, BM), :] = y2.astype(jnp.bfloat16)


def kernel(x, adj_norm, W0, W1, W2):
    w = jnp.stack([W0, W1, W2])
    out, _ = pl.pallas_call(
        _mega_kernel,
        grid=(3, NB),
        in_specs=[
            pl.BlockSpec((BM, N),
                         lambda l, i: (jnp.where(l == 0, i, NB - 1), 0)),
            pl.BlockSpec((N, D), lambda l, i: (0, 0)),
            pl.BlockSpec((3, D, D), lambda l, i: (0, 0, 0)),
        ],
        out_specs=[
            pl.BlockSpec((BM, D), lambda l, i: (i, l)),
            pl.BlockSpec(memory_space=pl.ANY),
        ],
        out_shape=[
            jax.ShapeDtypeStruct((N, 3 * D), jnp.float32),
            jax.ShapeDtypeStruct((N, N), jnp.bfloat16),
        ],
        scratch_shapes=[
            pltpu.VMEM((2, BM, N), jnp.bfloat16),
            pltpu.VMEM((2, N, D), jnp.bfloat16),
            pltpu.SemaphoreType.DMA((2,)),
            pltpu.SemaphoreType.DMA((2,)),
        ],
    )(adj_norm, x, w)
    return out


# speedup vs baseline: 1.0915x; 1.0915x over previous
import jax
import jax.numpy as jnp
from jax.experimental import pallas as pl
from jax.experimental.pallas import tpu as pltpu

N = 8192
D = 128
BM0 = 512
BM = 512
CB = 3


def _layer0_kernel(a_ref, x_ref, w0_ref, w1_ref, o_ref, a16_ref, y1_ref, y0_scr):
    @pl.when(pl.program_id(0) == 0)
    def _prologue():
        y0 = jnp.dot(x_ref[...], w0_ref[...], preferred_element_type=jnp.float32)
        y0_scr[...] = y0.astype(jnp.bfloat16)

    a16 = a_ref[...].astype(jnp.bfloat16)
    a16_ref[...] = a16
    h = jnp.maximum(
        jnp.dot(a16, y0_scr[...], preferred_element_type=jnp.float32), 0.0)
    o_ref[...] = h
    y1 = jnp.dot(h, w1_ref[...], preferred_element_type=jnp.float32)
    y1_ref[...] = y1.astype(jnp.bfloat16)


def _layer0(a, x, w0, w1):
    return pl.pallas_call(
        _layer0_kernel,
        grid=(N // BM0,),
        in_specs=[
            pl.BlockSpec((BM0, N), lambda i: (i, 0)),
            pl.BlockSpec((N, D), lambda i: (0, 0)),
            pl.BlockSpec((D, D), lambda i: (0, 0)),
            pl.BlockSpec((D, D), lambda i: (0, 0)),
        ],
        out_specs=[
            pl.BlockSpec((BM0, D), lambda i: (i, 0)),
            pl.BlockSpec((BM0, N), lambda i: (i, 0)),
            pl.BlockSpec((BM0, D), lambda i: (i, 0)),
        ],
        out_shape=[
            jax.ShapeDtypeStruct((N, 3 * D), jnp.float32),
            jax.ShapeDtypeStruct((N, N), jnp.bfloat16),
            jax.ShapeDtypeStruct((N, D), jnp.bfloat16),
        ],
        scratch_shapes=[pltpu.VMEM((N, D), jnp.bfloat16)],
    )(a, x, w0, w1)


def _layers12_kernel(a16_ref, y1_ref, w2_ref, o_in_ref, o_ref, y_scr, cache):
    del o_in_ref
    l = pl.program_id(0)
    i = pl.program_id(1)

    @pl.when((l == 0) & (i == 0))
    def _prologue():
        y_scr[0] = y1_ref[...]

    def _compute(av):
        h = jnp.maximum(
            jnp.dot(av, y_scr[l], preferred_element_type=jnp.float32), 0.0)
        o_ref[...] = h
        return h

    @pl.when(l == 0)
    def _phase0():
        av = a16_ref[...]

        @pl.when(i < CB)
        def _fill_cache():
            cache[pl.ds(i * BM, BM), :] = av

        h = _compute(av)
        y2 = jnp.dot(h, w2_ref[...], preferred_element_type=jnp.float32)
        y_scr[1, pl.ds(i * BM, BM), :] = y2.astype(jnp.bfloat16)

    @pl.when((l == 1) & (i < CB))
    def _phase1_cached():
        _compute(cache[pl.ds(i * BM, BM), :])

    @pl.when((l == 1) & (i >= CB))
    def _phase1_stream():
        _compute(a16_ref[...])


def _layers12(a16, y1, w2, o):
    return pl.pallas_call(
        _layers12_kernel,
        grid=(2, N // BM),
        in_specs=[
            pl.BlockSpec(
                (BM, N),
                lambda l, i: (
                    jnp.where(l == 0, i,
                              jnp.where(i < CB, N // BM - 1, i)), 0)),
            pl.BlockSpec((N, D), lambda l, i: (0, 0)),
            pl.BlockSpec((D, D), lambda l, i: (0, 0)),
            pl.BlockSpec(memory_space=pl.ANY),
        ],
        out_specs=pl.BlockSpec((BM, D), lambda l, i: (i, 1 + l)),
        out_shape=jax.ShapeDtypeStruct((N, 3 * D), jnp.float32),
        input_output_aliases={3: 0},
        scratch_shapes=[
            pltpu.VMEM((2, N, D), jnp.bfloat16),
            pltpu.VMEM((CB * BM, N), jnp.bfloat16),
        ],
    )(a16, y1, w2, o)


def kernel(x, adj_norm, W0, W1, W2):
    o1, a16, y1 = _layer0(adj_norm, x, W0, W1)
    return _layers12(a16, y1, W2, o1)


# speedup vs baseline: 1.1528x; 1.0562x over previous
import jax
import jax.numpy as jnp
from jax.experimental import pallas as pl
from jax.experimental.pallas import tpu as pltpu

N = 8192
D = 128
BM0 = 512
BM = 1024


def _layer0_kernel(a_ref, x_ref, w0_ref, w1_ref, o_ref, a16_ref, y1_ref, y0_scr):
    @pl.when(pl.program_id(0) == 0)
    def _prologue():
        y0 = jnp.dot(x_ref[...], w0_ref[...], preferred_element_type=jnp.float32)
        y0_scr[...] = y0.astype(jnp.bfloat16)

    a16 = a_ref[...].astype(jnp.bfloat16)
    a16_ref[...] = a16
    h = jnp.maximum(
        jnp.dot(a16, y0_scr[...], preferred_element_type=jnp.float32), 0.0)
    o_ref[...] = h
    y1 = jnp.dot(h, w1_ref[...], preferred_element_type=jnp.float32)
    y1_ref[...] = y1.astype(jnp.bfloat16)


def _layer0(a, x, w0, w1):
    return pl.pallas_call(
        _layer0_kernel,
        grid=(N // BM0,),
        in_specs=[
            pl.BlockSpec((BM0, N), lambda i: (i, 0)),
            pl.BlockSpec((N, D), lambda i: (0, 0)),
            pl.BlockSpec((D, D), lambda i: (0, 0)),
            pl.BlockSpec((D, D), lambda i: (0, 0)),
        ],
        out_specs=[
            pl.BlockSpec((BM0, D), lambda i: (i, 0)),
            pl.BlockSpec((BM0, N), lambda i: (i, 0)),
            pl.BlockSpec((BM0, D), lambda i: (i, 0)),
        ],
        out_shape=[
            jax.ShapeDtypeStruct((N, 3 * D), jnp.float32),
            jax.ShapeDtypeStruct((N, N), jnp.bfloat16),
            jax.ShapeDtypeStruct((N, D), jnp.bfloat16),
        ],
        scratch_shapes=[pltpu.VMEM((N, D), jnp.bfloat16)],
    )(a, x, w0, w1)


def _layers12_kernel(a16_ref, y1_ref, w2_ref, o_in_ref, o_ref, y_scr):
    del o_in_ref
    l = pl.program_id(0)
    i = pl.program_id(1)

    @pl.when((l == 0) & (i == 0))
    def _prologue():
        y_scr[0] = y1_ref[...]

    h = jnp.maximum(
        jnp.dot(a16_ref[...], y_scr[l], preferred_element_type=jnp.float32),
        0.0)
    o_ref[...] = h

    @pl.when(l == 0)
    def _epilogue():
        y2 = jnp.dot(h, w2_ref[...], preferred_element_type=jnp.float32)
        y_scr[1, pl.ds(i * BM, BM), :] = y2.astype(jnp.bfloat16)


def _layers12(a16, y1, w2, o):
    return pl.pallas_call(
        _layers12_kernel,
        grid=(2, N // BM),
        in_specs=[
            pl.BlockSpec((BM, N), lambda l, i: (i, 0)),
            pl.BlockSpec((N, D), lambda l, i: (0, 0)),
            pl.BlockSpec((D, D), lambda l, i: (0, 0)),
            pl.BlockSpec(memory_space=pl.ANY),
        ],
        out_specs=pl.BlockSpec((BM, D), lambda l, i: (i, 1 + l)),
        out_shape=jax.ShapeDtypeStruct((N, 3 * D), jnp.float32),
        input_output_aliases={3: 0},
        scratch_shapes=[pltpu.VMEM((2, N, D), jnp.bfloat16)],
    )(a16, y1, w2, o)


def kernel(x, adj_norm, W0, W1, W2):
    o1, a16, y1 = _layer0(adj_norm, x, W0, W1)
    return _layers12(a16, y1, W2, o1)


# speedup vs baseline: 1.1534x; 1.0006x over previous
import jax
import jax.numpy as jnp
from jax.experimental import pallas as pl
from jax.experimental.pallas import tpu as pltpu

N = 8192
D = 128
BM0 = 256
BM = 1024


def _layer0_kernel(a_ref, x_ref, w0_ref, w1_ref, o_ref, a16_ref, y1_ref, y0_scr):
    @pl.when(pl.program_id(0) == 0)
    def _prologue():
        y0 = jnp.dot(x_ref[...], w0_ref[...], preferred_element_type=jnp.float32)
        y0_scr[...] = y0.astype(jnp.bfloat16)

    a16 = a_ref[...].astype(jnp.bfloat16)
    a16_ref[...] = a16
    h = jnp.maximum(
        jnp.dot(a16, y0_scr[...], preferred_element_type=jnp.float32), 0.0)
    o_ref[...] = h
    y1 = jnp.dot(h, w1_ref[...], preferred_element_type=jnp.float32)
    y1_ref[...] = y1.astype(jnp.bfloat16)


def _layer0(a, x, w0, w1):
    return pl.pallas_call(
        _layer0_kernel,
        grid=(N // BM0,),
        in_specs=[
            pl.BlockSpec((BM0, N), lambda i: (i, 0)),
            pl.BlockSpec((N, D), lambda i: (0, 0)),
            pl.BlockSpec((D, D), lambda i: (0, 0)),
            pl.BlockSpec((D, D), lambda i: (0, 0)),
        ],
        out_specs=[
            pl.BlockSpec((BM0, D), lambda i: (i, 0)),
            pl.BlockSpec((BM0, N), lambda i: (i, 0)),
            pl.BlockSpec((BM0, D), lambda i: (i, 0)),
        ],
        out_shape=[
            jax.ShapeDtypeStruct((N, 3 * D), jnp.float32),
            jax.ShapeDtypeStruct((N, N), jnp.bfloat16),
            jax.ShapeDtypeStruct((N, D), jnp.bfloat16),
        ],
        scratch_shapes=[pltpu.VMEM((N, D), jnp.bfloat16)],
    )(a, x, w0, w1)


def _layers12_kernel(a16_ref, y1_ref, w2_ref, o_in_ref, o_ref, y_scr):
    del o_in_ref
    l = pl.program_id(0)
    i = pl.program_id(1)

    @pl.when((l == 0) & (i == 0))
    def _prologue():
        y_scr[0] = y1_ref[...]

    h = jnp.maximum(
        jnp.dot(a16_ref[...], y_scr[l], preferred_element_type=jnp.float32),
        0.0)
    o_ref[...] = h

    @pl.when(l == 0)
    def _epilogue():
        y2 = jnp.dot(h, w2_ref[...], preferred_element_type=jnp.float32)
        y_scr[1, pl.ds(i * BM, BM), :] = y2.astype(jnp.bfloat16)


def _layers12(a16, y1, w2, o):
    return pl.pallas_call(
        _layers12_kernel,
        grid=(2, N // BM),
        in_specs=[
            pl.BlockSpec((BM, N), lambda l, i: (i, 0)),
            pl.BlockSpec((N, D), lambda l, i: (0, 0)),
            pl.BlockSpec((D, D), lambda l, i: (0, 0)),
            pl.BlockSpec(memory_space=pl.ANY),
        ],
        out_specs=pl.BlockSpec((BM, D), lambda l, i: (i, 1 + l)),
        out_shape=jax.ShapeDtypeStruct((N, 3 * D), jnp.float32),
        input_output_aliases={3: 0},
        scratch_shapes=[pltpu.VMEM((2, N, D), jnp.bfloat16)],
    )(a16, y1, w2, o)


def kernel(x, adj_norm, W0, W1, W2):
    o1, a16, y1 = _layer0(adj_norm, x, W0, W1)
    return _layers12(a16, y1, W2, o1)
